# baseline (speedup 1.0000x reference)
import functools

import jax
import jax.numpy as jnp
from jax.experimental import pallas as pl
from jax.experimental.pallas import tpu as pltpu

_LOG2E = 1.4426950408889634


def _gat_block(nodes_ref, nbr_ref, w_ref, wa_src_ref, wa_tgt_ref, bias_ref,
               out_ref):
    w = w_ref[...]
    wa_src = wa_src_ref[...]
    wa_tgt = wa_tgt_ref[...]

    nodes = nodes_ref[...]
    nbr = nbr_ref[...]

    s_t = jnp.sum(nodes * wa_tgt, axis=-1)
    s_s = jnp.sum(nbr * wa_src[None], axis=-1)

    s = s_s + s_t[:, None]
    e = jnp.exp2(jnp.maximum(s, 0.2 * s))
    denom = jnp.sum(e, axis=1)
    num = jnp.sum(nbr * e[..., None], axis=1)

    out = jnp.dot(num, w.T, preferred_element_type=jnp.float32)
    out = out / (denom[:, None] + 1e-16) + bias_ref[...]
    out_ref[...] = jnp.where(out > 0.0, out, jnp.exp(out) - 1.0)


@functools.partial(jax.jit, static_argnames=())
def kernel(nodes, neighbors, W, a_src, a_tgt, bias):
    n, f_in = nodes.shape
    deg = neighbors.shape[1]
    f_out = W.shape[0]
    bn = 1000
    grid = (n // bn,)
    bias2 = bias.reshape(1, f_out)
    wa_src2 = (a_src.reshape(1, f_out) @ W) * _LOG2E
    wa_tgt2 = (a_tgt.reshape(1, f_out) @ W) * _LOG2E
    return pl.pallas_call(
        _gat_block,
        grid=grid,
        in_specs=[
            pl.BlockSpec((bn, f_in), lambda i: (i, 0)),
            pl.BlockSpec((bn, deg, f_in), lambda i: (i, 0, 0)),
            pl.BlockSpec((f_out, f_in), lambda i: (0, 0)),
            pl.BlockSpec((1, f_in), lambda i: (0, 0)),
            pl.BlockSpec((1, f_in), lambda i: (0, 0)),
            pl.BlockSpec((1, f_out), lambda i: (0, 0)),
        ],
        out_specs=pl.BlockSpec((bn, f_out), lambda i: (i, 0)),
        out_shape=jax.ShapeDtypeStruct((n, f_out), jnp.float32),
        compiler_params=pltpu.CompilerParams(
            dimension_semantics=("parallel",)),
    )(nodes, neighbors, W, wa_src2, wa_tgt2, bias2)


# speedup vs baseline: 1.0377x; 1.0377x over previous
import functools

import jax
import jax.numpy as jnp
from jax.experimental import pallas as pl
from jax.experimental.pallas import tpu as pltpu

_LOG2E = 1.4426950408889634


def _gat_block(nodes_ref, nbr_ref, w_ref, a_src_ref, a_tgt_ref, bias_ref,
               out_ref):
    w = w_ref[...]
    wa_src = jnp.dot(a_src_ref[0], w, preferred_element_type=jnp.float32)
    wa_tgt = jnp.dot(a_tgt_ref[0], w, preferred_element_type=jnp.float32)
    wa_src = wa_src * _LOG2E
    wa_tgt = wa_tgt * _LOG2E

    nodes = nodes_ref[...]
    nbr = nbr_ref[...]

    s_t = jnp.sum(nodes * wa_tgt, axis=-1)
    s_s = jnp.sum(nbr * wa_src[None], axis=-1)

    s = s_s + s_t[:, None]
    e = jnp.exp2(jnp.maximum(s, 0.2 * s))
    denom = jnp.sum(e, axis=1)
    num = jnp.sum(nbr * e[..., None], axis=1)

    out = jnp.dot(num, w.T, preferred_element_type=jnp.float32)
    out = out / (denom[:, None] + 1e-16) + bias_ref[...]
    out_ref[...] = jnp.where(out > 0.0, out, jnp.exp(out) - 1.0)


@functools.partial(jax.jit, static_argnames=())
def kernel(nodes, neighbors, W, a_src, a_tgt, bias):
    n, f_in = nodes.shape
    deg = neighbors.shape[1]
    f_out = W.shape[0]
    bn = 1000
    grid = (n // bn,)
    bias2 = bias.reshape(1, f_out)
    return pl.pallas_call(
        _gat_block,
        grid=grid,
        in_specs=[
            pl.BlockSpec((bn, f_in), lambda i: (i, 0)),
            pl.BlockSpec((bn, deg, f_in), lambda i: (i, 0, 0)),
            pl.BlockSpec((f_out, f_in), lambda i: (0, 0)),
            pl.BlockSpec((1, 1, f_out), lambda i: (0, 0, 0)),
            pl.BlockSpec((1, 1, f_out), lambda i: (0, 0, 0)),
            pl.BlockSpec((1, f_out), lambda i: (0, 0)),
        ],
        out_specs=pl.BlockSpec((bn, f_out), lambda i: (i, 0)),
        out_shape=jax.ShapeDtypeStruct((n, f_out), jnp.float32),
        compiler_params=pltpu.CompilerParams(
            dimension_semantics=("parallel",)),
    )(nodes, neighbors, W, a_src, a_tgt, bias2)
